# Initial kernel scaffold; baseline (speedup 1.0000x reference)
import jax
import jax.numpy as jnp
from jax.experimental import pallas as pl


def kernel(block_features, block_edge_index, block_edge_attr, cons_block_id, vars_block_id, c_sub_owned, v_sub_owned, cons_boundary_feat, vars_boundary_feat, params):
    raise NotImplementedError("write your pallas kernel here")


# speedup vs baseline: 1.0721x; 1.0721x over previous
import functools

import jax
import jax.numpy as jnp
from jax.experimental import pallas as pl
from jax.experimental.pallas import tpu as pltpu

H = 4
C = 32


def _ln(x, g, b, eps=1e-5):
    m = jnp.mean(x, axis=-1, keepdims=True)
    v = jnp.var(x, axis=-1, keepdims=True)
    return (x - m) / jnp.sqrt(v + eps) * g + b


_TR = 512


def _composer_body(a_ref, z_ref, f_ref, w1a_ref, w1b_ref, w1c_ref, b1_ref,
                   g1_ref, n1_ref, w2_ref, b2_ref, g2_ref, n2_ref,
                   w3_ref, b3_ref, o_ref):
    t = (jnp.dot(a_ref[...], w1a_ref[...], preferred_element_type=jnp.float32)
         + jnp.dot(z_ref[...], w1b_ref[...], preferred_element_type=jnp.float32)
         + jnp.dot(f_ref[...], w1c_ref[...], preferred_element_type=jnp.float32)
         + b1_ref[...])
    h = jax.nn.relu(_ln(t, g1_ref[...], n1_ref[...]))
    t2 = jnp.dot(h, w2_ref[...], preferred_element_type=jnp.float32) + b2_ref[...]
    h2 = jax.nn.relu(_ln(t2, g2_ref[...], n2_ref[...]))
    o_ref[...] = jnp.dot(h2, w3_ref[...], preferred_element_type=jnp.float32) + b3_ref[...]


def _composer_pallas(a, z, f, p):
    n = a.shape[0]
    grid = pl.cdiv(n, _TR)
    dh = p['W1'].shape[1]
    dout = p['W3'].shape[1]
    w1a = p['W1'][:128]
    w1b = p['W1'][128:256]
    w1c = p['W1'][256:]
    row = lambda i: (i, 0)
    rep = lambda i: (0, 0)
    out = pl.pallas_call(
        _composer_body,
        grid=(grid,),
        in_specs=[
            pl.BlockSpec((_TR, 128), row),
            pl.BlockSpec((_TR, 128), row),
            pl.BlockSpec((_TR, 16), row),
            pl.BlockSpec((128, dh), rep),
            pl.BlockSpec((128, dh), rep),
            pl.BlockSpec((16, dh), rep),
            pl.BlockSpec((1, dh), rep),
            pl.BlockSpec((1, dh), rep),
            pl.BlockSpec((1, dh), rep),
            pl.BlockSpec((dh, dh), rep),
            pl.BlockSpec((1, dh), rep),
            pl.BlockSpec((1, dh), rep),
            pl.BlockSpec((1, dh), rep),
            pl.BlockSpec((dh, dout), rep),
            pl.BlockSpec((1, dout), rep),
        ],
        out_specs=pl.BlockSpec((_TR, dout), row),
        out_shape=jax.ShapeDtypeStruct((n, dout), jnp.float32),
    )(a, z, f, w1a, w1b, w1c, p['b1'][None], p['g1'][None], p['bn1'][None],
      p['W2'], p['b2'][None], p['g2'][None], p['bn2'][None], p['W3'], p['b3'][None])
    return out


def _gatv2(x, src, dst, e, p):
    N = x.shape[0]
    E = src.shape[0]
    xl = x @ p['Wl'] + p['bl']
    xr = x @ p['Wr'] + p['br']
    ep = e @ p['We']
    m = (xl[src] + xr[dst] + ep).reshape(E, H, C)
    m = jax.nn.leaky_relu(m, 0.2)
    alpha = jnp.einsum('ehc,hc->eh', m, p['att'])
    ex = jnp.exp(alpha)
    den = jax.ops.segment_sum(ex, dst, num_segments=N)
    msg = xl[src].reshape(E, H, C) * ex[:, :, None]
    out = jax.ops.segment_sum(msg, dst, num_segments=N).reshape(N, H * C)
    out = out / (den + 1e-16).repeat(C, axis=-1)
    return out + p['bias']


def _block_gnn(bf, ei, ea, p):
    x = _ln(bf, p['in_ln_g'], p['in_ln_b'])
    x = jax.nn.relu(x @ p['in_W'] + p['in_b'])
    e = jax.nn.relu(ea @ p['e_W'] + p['e_b'])
    src = jnp.concatenate([ei[0], ei[1]])
    dst = jnp.concatenate([ei[1], ei[0]])
    ef = jnp.concatenate([e, e], axis=0)
    x = x + _gatv2(x, src, dst, ef, p['conv1'])
    x = _ln(x, p['n1_g'], p['n1_b'])
    x = x + _gatv2(x, src, dst, ef, p['conv2'])
    x = _ln(x, p['n2_g'], p['n2_b'])
    x = jax.nn.relu(x @ p['o_W1'] + p['o_b1'])
    return x @ p['o_W2'] + p['o_b2']


def kernel(block_features, block_edge_index, block_edge_attr, cons_block_id,
           vars_block_id, c_sub_owned, v_sub_owned, cons_boundary_feat,
           vars_boundary_feat, params):
    z = _block_gnn(block_features, block_edge_index, block_edge_attr, params['gnn'])
    zc = z[cons_block_id]
    zv = z[vars_block_id]
    co = _composer_pallas(c_sub_owned, zc, cons_boundary_feat, params['cons'])
    vo = _composer_pallas(v_sub_owned, zv, vars_boundary_feat, params['var'])
    return (co, vo)


# speedup vs baseline: 12.8653x; 11.9996x over previous
import functools

import jax
import jax.numpy as jnp
from jax import lax
from jax.experimental import pallas as pl
from jax.experimental.pallas import tpu as pltpu
from jax.experimental.pallas import tpu_sc as plsc

H = 4
C = 32

NC = 2
NS = 16
NW = NC * NS

K = 10000
RT = 10240
DUMMY = 10000
E = 160000
EPAD = 163840
PW = EPAD // NW
CHG = 128


def _ln(x, g, b, eps=1e-5):
    m = jnp.mean(x, axis=-1, keepdims=True)
    v = jnp.var(x, axis=-1, keepdims=True)
    return (x - m) / jnp.sqrt(v + eps) * g + b


def _egather_body(xlr_hbm, src_hbm, dst_hbm, a_hbm, b_hbm,
                  sidx, didx, rowsa, rowsb, sem):
    c = lax.axis_index("c")
    s = lax.axis_index("s")
    w = s * NC + c

    def _chunk(k, _):
        base = w * PW + k * CHG
        pltpu.sync_copy(src_hbm.at[pl.ds(base, CHG)], sidx)
        pltpu.sync_copy(dst_hbm.at[pl.ds(base, CHG)], didx)
        cpa = pltpu.async_copy(xlr_hbm.at[sidx], rowsa, sem)
        cpb = pltpu.async_copy(xlr_hbm.at[didx], rowsb, sem)
        cpa.wait()
        cpb.wait()
        pltpu.sync_copy(rowsa, a_hbm.at[pl.ds(base, CHG)])
        pltpu.sync_copy(rowsb, b_hbm.at[pl.ds(base, CHG)])
        return 0
    lax.fori_loop(0, PW // CHG, _chunk, 0)


_egather = functools.partial(
    pl.kernel, _egather_body,
    mesh=plsc.VectorSubcoreMesh(core_axis_name="c", subcore_axis_name="s"),
    out_type=(jax.ShapeDtypeStruct((EPAD, 256), jnp.float32),
              jax.ShapeDtypeStruct((EPAD, 256), jnp.float32)),
    scratch_types=[
        pltpu.VMEM((CHG,), jnp.int32),
        pltpu.VMEM((CHG,), jnp.int32),
        pltpu.VMEM((CHG, 256), jnp.float32),
        pltpu.VMEM((CHG, 256), jnp.float32),
        pltpu.SemaphoreType.DMA,
    ],
)


def _make_zgather(n_pad, chunk, nchunk):
    def body(z_hbm, idx_hbm, out_hbm, idxv, rows, sem):
        c = lax.axis_index("c")
        s = lax.axis_index("s")
        w = s * NC + c
        per_w = n_pad // NW

        def _chunk(k, _):
            base = w * per_w + k * chunk
            pltpu.sync_copy(idx_hbm.at[pl.ds(base, chunk)], idxv)
            pltpu.async_copy(z_hbm.at[idxv], rows, sem).wait()
            pltpu.sync_copy(rows, out_hbm.at[pl.ds(base, chunk)])
            return 0
        lax.fori_loop(0, nchunk, _chunk, 0)

    return functools.partial(
        pl.kernel, body,
        mesh=plsc.VectorSubcoreMesh(core_axis_name="c", subcore_axis_name="s"),
        out_type=jax.ShapeDtypeStruct((n_pad, 128), jnp.float32),
        scratch_types=[
            pltpu.VMEM((chunk,), jnp.int32),
            pltpu.VMEM((chunk, 128), jnp.float32),
            pltpu.SemaphoreType.DMA,
        ],
    )


ZPAD = 150528
_zgather = _make_zgather(ZPAD, 112, 42)


_TE = 512


def _edge_body(a_ref, b_ref, ea_ref, ew_ref, ebias_ref, we_ref, att_ref,
               exp_ref, wmf_ref, wmb_ref, exf_ref, exb_ref):
    e = jax.nn.relu(jnp.dot(ea_ref[...], ew_ref[...],
                            preferred_element_type=jnp.float32) + ebias_ref[...])
    ep = jnp.dot(e, we_ref[...], preferred_element_type=jnp.float32)
    av = a_ref[...]
    bv = b_ref[...]
    al = av[:, :128]
    ar = av[:, 128:]
    bl = bv[:, :128]
    br = bv[:, 128:]
    mf = al + br + ep
    mb = bl + ar + ep
    lf = jnp.where(mf > 0, mf, 0.2 * mf)
    lb = jnp.where(mb > 0, mb, 0.2 * mb)
    exf = jnp.exp(jnp.dot(lf, att_ref[...], preferred_element_type=jnp.float32))
    exb = jnp.exp(jnp.dot(lb, att_ref[...], preferred_element_type=jnp.float32))
    wmf_ref[...] = al * jnp.dot(exf, exp_ref[...], preferred_element_type=jnp.float32)
    wmb_ref[...] = bl * jnp.dot(exb, exp_ref[...], preferred_element_type=jnp.float32)
    exf_ref[...] = exf
    exb_ref[...] = exb


def _edge_pallas(a, b, ea_p, att16, expand16, e_W, e_b, p):
    grid = EPAD // _TE
    row = lambda i: (i, 0)
    rep = lambda i: (0, 0)
    return pl.pallas_call(
        _edge_body,
        grid=(grid,),
        in_specs=[
            pl.BlockSpec((_TE, 256), row),
            pl.BlockSpec((_TE, 256), row),
            pl.BlockSpec((_TE, 4), row),
            pl.BlockSpec((4, 128), rep),
            pl.BlockSpec((1, 128), rep),
            pl.BlockSpec((128, 128), rep),
            pl.BlockSpec((128, 16), rep),
            pl.BlockSpec((16, 128), rep),
        ],
        out_specs=[
            pl.BlockSpec((_TE, 128), row),
            pl.BlockSpec((_TE, 128), row),
            pl.BlockSpec((_TE, 16), row),
            pl.BlockSpec((_TE, 16), row),
        ],
        out_shape=[
            jax.ShapeDtypeStruct((EPAD, 128), jnp.float32),
            jax.ShapeDtypeStruct((EPAD, 128), jnp.float32),
            jax.ShapeDtypeStruct((EPAD, 16), jnp.float32),
            jax.ShapeDtypeStruct((EPAD, 16), jnp.float32),
        ],
    )(a, b, ea_p, e_W, e_b[None], p['We'], att16, expand16)


_TR = 512


def _composer_body(a_ref, z_ref, f_ref, w1a_ref, w1b_ref, w1c_ref, b1_ref,
                   g1_ref, n1_ref, w2_ref, b2_ref, g2_ref, n2_ref,
                   w3_ref, b3_ref, o_ref):
    t = (jnp.dot(a_ref[...], w1a_ref[...], preferred_element_type=jnp.float32)
         + jnp.dot(z_ref[...], w1b_ref[...], preferred_element_type=jnp.float32)
         + jnp.dot(f_ref[...], w1c_ref[...], preferred_element_type=jnp.float32)
         + b1_ref[...])
    h = jax.nn.relu(_ln(t, g1_ref[...], n1_ref[...]))
    t2 = jnp.dot(h, w2_ref[...], preferred_element_type=jnp.float32) + b2_ref[...]
    h2 = jax.nn.relu(_ln(t2, g2_ref[...], n2_ref[...]))
    o_ref[...] = jnp.dot(h2, w3_ref[...], preferred_element_type=jnp.float32) + b3_ref[...]


def _composer_pallas(a, z, f, p):
    n = a.shape[0]
    grid = pl.cdiv(n, _TR)
    dh = p['W1'].shape[1]
    dout = p['W3'].shape[1]
    w1a = p['W1'][:128]
    w1b = p['W1'][128:256]
    w1c = p['W1'][256:]
    row = lambda i: (i, 0)
    rep = lambda i: (0, 0)
    out = pl.pallas_call(
        _composer_body,
        grid=(grid,),
        in_specs=[
            pl.BlockSpec((_TR, 128), row),
            pl.BlockSpec((_TR, 128), row),
            pl.BlockSpec((_TR, 16), row),
            pl.BlockSpec((128, dh), rep),
            pl.BlockSpec((128, dh), rep),
            pl.BlockSpec((16, dh), rep),
            pl.BlockSpec((1, dh), rep),
            pl.BlockSpec((1, dh), rep),
            pl.BlockSpec((1, dh), rep),
            pl.BlockSpec((dh, dh), rep),
            pl.BlockSpec((1, dh), rep),
            pl.BlockSpec((1, dh), rep),
            pl.BlockSpec((1, dh), rep),
            pl.BlockSpec((dh, dout), rep),
            pl.BlockSpec((1, dout), rep),
        ],
        out_specs=pl.BlockSpec((_TR, dout), row),
        out_shape=jax.ShapeDtypeStruct((n, dout), jnp.float32),
    )(a, z, f, w1a, w1b, w1c, p['b1'][None], p['g1'][None], p['bn1'][None],
      p['W2'], p['b2'][None], p['g2'][None], p['bn2'][None], p['W3'], p['b3'][None])
    return out


def _pad_rows(x, rows):
    return jnp.zeros((rows, x.shape[1]), x.dtype).at[:x.shape[0]].set(x)


def _gatv2_sc(x, srcp, dstp, ea_p, att16, expand16, e_W, e_b, p):
    xl = x @ p['Wl'] + p['bl']
    xr = x @ p['Wr'] + p['br']
    xlr = _pad_rows(jnp.concatenate([xl, xr], axis=1), RT)
    a, b = _egather()(xlr, srcp, dstp)
    wmf, wmb, exf, exb = _edge_pallas(a, b, ea_p, att16, expand16, e_W, e_b, p)
    msg = (jax.ops.segment_sum(wmf, dstp, num_segments=RT)
           + jax.ops.segment_sum(wmb, srcp, num_segments=RT))[:K]
    den = (jax.ops.segment_sum(exf[:, :4], dstp, num_segments=RT)
           + jax.ops.segment_sum(exb[:, :4], srcp, num_segments=RT))[:K]
    out = msg / jnp.repeat(den + 1e-16, C, axis=-1)
    return out + p['bias']


def _att_mats(att):
    rows = jnp.arange(128)
    cols = jnp.repeat(jnp.arange(4), 32)
    att16 = jnp.zeros((128, 16), jnp.float32).at[rows, cols].set(att.reshape(-1))
    expand16 = jnp.zeros((16, 128), jnp.float32).at[cols, rows].set(1.0)
    return att16, expand16


def _block_gnn(bf, ei, ea, p):
    x = _ln(bf, p['in_ln_g'], p['in_ln_b'])
    x = jax.nn.relu(x @ p['in_W'] + p['in_b'])
    pad_idx = jnp.full((EPAD - E,), DUMMY, jnp.int32)
    srcp = jnp.concatenate([ei[0].astype(jnp.int32), pad_idx])
    dstp = jnp.concatenate([ei[1].astype(jnp.int32), pad_idx])
    ea_p = jnp.zeros((EPAD, 4), jnp.float32).at[:E].set(ea)
    a1, e1 = _att_mats(p['conv1']['att'])
    a2, e2 = _att_mats(p['conv2']['att'])
    x = x + _gatv2_sc(x, srcp, dstp, ea_p, a1, e1, p['e_W'], p['e_b'], p['conv1'])
    x = _ln(x, p['n1_g'], p['n1_b'])
    x = x + _gatv2_sc(x, srcp, dstp, ea_p, a2, e2, p['e_W'], p['e_b'], p['conv2'])
    x = _ln(x, p['n2_g'], p['n2_b'])
    x = jax.nn.relu(x @ p['o_W1'] + p['o_b1'])
    return x @ p['o_W2'] + p['o_b2']


def kernel(block_features, block_edge_index, block_edge_attr, cons_block_id,
           vars_block_id, c_sub_owned, v_sub_owned, cons_boundary_feat,
           vars_boundary_feat, params):
    z = _block_gnn(block_features, block_edge_index, block_edge_attr, params['gnn'])
    nc = cons_block_id.shape[0]
    nv = vars_block_id.shape[0]
    idx = jnp.zeros((ZPAD,), jnp.int32)
    idx = idx.at[:nc].set(cons_block_id.astype(jnp.int32))
    idx = idx.at[50176:50176 + nv].set(vars_block_id.astype(jnp.int32))
    zg = _zgather()(z, idx)
    zc = zg[:nc]
    zv = zg[50176:50176 + nv]
    co = _composer_pallas(c_sub_owned, zc, cons_boundary_feat, params['cons'])
    vo = _composer_pallas(v_sub_owned, zv, vars_boundary_feat, params['var'])
    return (co, vo)


# speedup vs baseline: 15.1696x; 1.1791x over previous
import functools

import jax
import jax.numpy as jnp
from jax import lax
from jax.experimental import pallas as pl
from jax.experimental.pallas import tpu as pltpu
from jax.experimental.pallas import tpu_sc as plsc

H = 4
C = 32

NC = 2
NS = 16
NW = NC * NS

K = 10000
RT = 10240
DUMMY = 10000
E = 160000
EPAD = 163840
PW = EPAD // NW
CHG = 128


def _ln(x, g, b, eps=1e-5):
    m = jnp.mean(x, axis=-1, keepdims=True)
    v = jnp.var(x, axis=-1, keepdims=True)
    return (x - m) / jnp.sqrt(v + eps) * g + b


def _egather_body(xlr_hbm, src_hbm, dst_hbm, a_hbm, b_hbm,
                  sidx, didx, rowsa, rowsb, sem):
    c = lax.axis_index("c")
    s = lax.axis_index("s")
    w = s * NC + c

    def _chunk(k, _):
        base = w * PW + k * CHG
        pltpu.sync_copy(src_hbm.at[pl.ds(base, CHG)], sidx)
        pltpu.sync_copy(dst_hbm.at[pl.ds(base, CHG)], didx)
        cpa = pltpu.async_copy(xlr_hbm.at[sidx], rowsa, sem)
        cpb = pltpu.async_copy(xlr_hbm.at[didx], rowsb, sem)
        cpa.wait()
        cpb.wait()
        pltpu.sync_copy(rowsa, a_hbm.at[pl.ds(base, CHG)])
        pltpu.sync_copy(rowsb, b_hbm.at[pl.ds(base, CHG)])
        return 0
    lax.fori_loop(0, PW // CHG, _chunk, 0)


_egather = functools.partial(
    pl.kernel, _egather_body,
    mesh=plsc.VectorSubcoreMesh(core_axis_name="c", subcore_axis_name="s"),
    out_type=(jax.ShapeDtypeStruct((EPAD, 256), jnp.float32),
              jax.ShapeDtypeStruct((EPAD, 256), jnp.float32)),
    scratch_types=[
        pltpu.VMEM((CHG,), jnp.int32),
        pltpu.VMEM((CHG,), jnp.int32),
        pltpu.VMEM((CHG, 256), jnp.float32),
        pltpu.VMEM((CHG, 256), jnp.float32),
        pltpu.SemaphoreType.DMA,
    ],
)


def _make_zgather(n_pad, chunk, nchunk):
    def body(z_hbm, idx_hbm, out_hbm, idxv, rows, sem):
        c = lax.axis_index("c")
        s = lax.axis_index("s")
        w = s * NC + c
        per_w = n_pad // NW

        def _chunk(k, _):
            base = w * per_w + k * chunk
            pltpu.sync_copy(idx_hbm.at[pl.ds(base, chunk)], idxv)
            pltpu.async_copy(z_hbm.at[idxv], rows, sem).wait()
            pltpu.sync_copy(rows, out_hbm.at[pl.ds(base, chunk)])
            return 0
        lax.fori_loop(0, nchunk, _chunk, 0)

    return functools.partial(
        pl.kernel, body,
        mesh=plsc.VectorSubcoreMesh(core_axis_name="c", subcore_axis_name="s"),
        out_type=jax.ShapeDtypeStruct((n_pad, 128), jnp.float32),
        scratch_types=[
            pltpu.VMEM((chunk,), jnp.int32),
            pltpu.VMEM((chunk, 128), jnp.float32),
            pltpu.SemaphoreType.DMA,
        ],
    )


ZPAD = 150528
_zgather = _make_zgather(ZPAD, 112, 42)


_TE = 512


def _edge_body(a_ref, b_ref, ea_ref, ew_ref, ebias_ref, we_ref, att_ref,
               exp_ref, wmf_ref, wmb_ref):
    e = jax.nn.relu(jnp.dot(ea_ref[...], ew_ref[...],
                            preferred_element_type=jnp.float32) + ebias_ref[...])
    ep = jnp.dot(e, we_ref[...], preferred_element_type=jnp.float32)
    av = a_ref[...]
    bv = b_ref[...]
    al = av[:, :128]
    ar = av[:, 128:]
    bl = bv[:, :128]
    br = bv[:, 128:]
    mf = al + br + ep
    mb = bl + ar + ep
    lf = jnp.where(mf > 0, mf, 0.2 * mf)
    lb = jnp.where(mb > 0, mb, 0.2 * mb)
    exf = jnp.exp(jnp.dot(lf, att_ref[...], preferred_element_type=jnp.float32))
    exb = jnp.exp(jnp.dot(lb, att_ref[...], preferred_element_type=jnp.float32))
    wmf_ref[:, :128] = al * jnp.dot(exf, exp_ref[...], preferred_element_type=jnp.float32)
    wmf_ref[:, 128:] = exf
    wmb_ref[:, :128] = bl * jnp.dot(exb, exp_ref[...], preferred_element_type=jnp.float32)
    wmb_ref[:, 128:] = exb


def _edge_pallas(a, b, ea_p, att16, expand16, e_W, e_b, p):
    grid = EPAD // _TE
    row = lambda i: (i, 0)
    rep = lambda i: (0, 0)
    return pl.pallas_call(
        _edge_body,
        grid=(grid,),
        in_specs=[
            pl.BlockSpec((_TE, 256), row),
            pl.BlockSpec((_TE, 256), row),
            pl.BlockSpec((_TE, 4), row),
            pl.BlockSpec((4, 128), rep),
            pl.BlockSpec((1, 128), rep),
            pl.BlockSpec((128, 128), rep),
            pl.BlockSpec((128, 16), rep),
            pl.BlockSpec((16, 128), rep),
        ],
        out_specs=[
            pl.BlockSpec((_TE, 144), row),
            pl.BlockSpec((_TE, 144), row),
        ],
        out_shape=[
            jax.ShapeDtypeStruct((EPAD, 144), jnp.float32),
            jax.ShapeDtypeStruct((EPAD, 144), jnp.float32),
        ],
    )(a, b, ea_p, e_W, e_b[None], p['We'], att16, expand16)


_TR = 512


def _composer_body(a_ref, z_ref, f_ref, w1a_ref, w1b_ref, w1c_ref, b1_ref,
                   g1_ref, n1_ref, w2_ref, b2_ref, g2_ref, n2_ref,
                   w3_ref, b3_ref, o_ref):
    t = (jnp.dot(a_ref[...], w1a_ref[...], preferred_element_type=jnp.float32)
         + jnp.dot(z_ref[...], w1b_ref[...], preferred_element_type=jnp.float32)
         + jnp.dot(f_ref[...], w1c_ref[...], preferred_element_type=jnp.float32)
         + b1_ref[...])
    h = jax.nn.relu(_ln(t, g1_ref[...], n1_ref[...]))
    t2 = jnp.dot(h, w2_ref[...], preferred_element_type=jnp.float32) + b2_ref[...]
    h2 = jax.nn.relu(_ln(t2, g2_ref[...], n2_ref[...]))
    o_ref[...] = jnp.dot(h2, w3_ref[...], preferred_element_type=jnp.float32) + b3_ref[...]


def _composer_pallas(a, z, f, p):
    n = a.shape[0]
    grid = pl.cdiv(n, _TR)
    dh = p['W1'].shape[1]
    dout = p['W3'].shape[1]
    w1a = p['W1'][:128]
    w1b = p['W1'][128:256]
    w1c = p['W1'][256:]
    row = lambda i: (i, 0)
    rep = lambda i: (0, 0)
    out = pl.pallas_call(
        _composer_body,
        grid=(grid,),
        in_specs=[
            pl.BlockSpec((_TR, 128), row),
            pl.BlockSpec((_TR, 128), row),
            pl.BlockSpec((_TR, 16), row),
            pl.BlockSpec((128, dh), rep),
            pl.BlockSpec((128, dh), rep),
            pl.BlockSpec((16, dh), rep),
            pl.BlockSpec((1, dh), rep),
            pl.BlockSpec((1, dh), rep),
            pl.BlockSpec((1, dh), rep),
            pl.BlockSpec((dh, dh), rep),
            pl.BlockSpec((1, dh), rep),
            pl.BlockSpec((1, dh), rep),
            pl.BlockSpec((1, dh), rep),
            pl.BlockSpec((dh, dout), rep),
            pl.BlockSpec((1, dout), rep),
        ],
        out_specs=pl.BlockSpec((_TR, dout), row),
        out_shape=jax.ShapeDtypeStruct((n, dout), jnp.float32),
    )(a, z, f, w1a, w1b, w1c, p['b1'][None], p['g1'][None], p['bn1'][None],
      p['W2'], p['b2'][None], p['g2'][None], p['bn2'][None], p['W3'], p['b3'][None])
    return out


def _pad_rows(x, rows):
    return jnp.zeros((rows, x.shape[1]), x.dtype).at[:x.shape[0]].set(x)


def _gatv2_sc(x, srcp, dstp, ea_p, att16, expand16, e_W, e_b, p):
    xl = x @ p['Wl'] + p['bl']
    xr = x @ p['Wr'] + p['br']
    xlr = _pad_rows(jnp.concatenate([xl, xr], axis=1), RT)
    a, b = _egather()(xlr, srcp, dstp)
    wmf, wmb = _edge_pallas(a, b, ea_p, att16, expand16, e_W, e_b, p)
    acc = (jax.ops.segment_sum(wmf, dstp, num_segments=RT)
           + jax.ops.segment_sum(wmb, srcp, num_segments=RT))[:K]
    msg = acc[:, :128]
    den = acc[:, 128:132]
    out = msg / jnp.repeat(den + 1e-16, C, axis=-1)
    return out + p['bias']


def _att_mats(att):
    rows = jnp.arange(128)
    cols = jnp.repeat(jnp.arange(4), 32)
    att16 = jnp.zeros((128, 16), jnp.float32).at[rows, cols].set(att.reshape(-1))
    expand16 = jnp.zeros((16, 128), jnp.float32).at[cols, rows].set(1.0)
    return att16, expand16


def _block_gnn(bf, ei, ea, p):
    x = _ln(bf, p['in_ln_g'], p['in_ln_b'])
    x = jax.nn.relu(x @ p['in_W'] + p['in_b'])
    pad_idx = jnp.full((EPAD - E,), DUMMY, jnp.int32)
    srcp = jnp.concatenate([ei[0].astype(jnp.int32), pad_idx])
    dstp = jnp.concatenate([ei[1].astype(jnp.int32), pad_idx])
    ea_p = jnp.zeros((EPAD, 4), jnp.float32).at[:E].set(ea)
    a1, e1 = _att_mats(p['conv1']['att'])
    a2, e2 = _att_mats(p['conv2']['att'])
    x = x + _gatv2_sc(x, srcp, dstp, ea_p, a1, e1, p['e_W'], p['e_b'], p['conv1'])
    x = _ln(x, p['n1_g'], p['n1_b'])
    x = x + _gatv2_sc(x, srcp, dstp, ea_p, a2, e2, p['e_W'], p['e_b'], p['conv2'])
    x = _ln(x, p['n2_g'], p['n2_b'])
    x = jax.nn.relu(x @ p['o_W1'] + p['o_b1'])
    return x @ p['o_W2'] + p['o_b2']


def kernel(block_features, block_edge_index, block_edge_attr, cons_block_id,
           vars_block_id, c_sub_owned, v_sub_owned, cons_boundary_feat,
           vars_boundary_feat, params):
    z = _block_gnn(block_features, block_edge_index, block_edge_attr, params['gnn'])
    nc = cons_block_id.shape[0]
    nv = vars_block_id.shape[0]
    idx = jnp.zeros((ZPAD,), jnp.int32)
    idx = idx.at[:nc].set(cons_block_id.astype(jnp.int32))
    idx = idx.at[50176:50176 + nv].set(vars_block_id.astype(jnp.int32))
    zg = _zgather()(z, idx)
    zc = zg[:nc]
    zv = zg[50176:50176 + nv]
    co = _composer_pallas(c_sub_owned, zc, cons_boundary_feat, params['cons'])
    vo = _composer_pallas(v_sub_owned, zv, vars_boundary_feat, params['var'])
    return (co, vo)
